# baseline (speedup 1.0000x reference)
import functools

import jax
import jax.numpy as jnp
from jax import lax
from jax.experimental import pallas as pl
from jax.experimental.pallas import tpu as pltpu
from jax.experimental.pallas import tpu_sc as plsc

ENT = 10000
NENT = 10006
NREL = 500
NH = 20000
NE = 320000
D = 128
B = 4096
L = 16
NC, NS = 2, 16
NW = NC * NS
CH = 128
HCH = 5
NH_PAD = NW * HCH * CH
CHB = 64
PCH = 320
DEPTH = 4
GRP = 8
NGRP = PCH // GRP
NCHK = (NW // 2) * PCH
NE_PAD = NCHK * CHB
PCHA = 2 * HCH
HALF = 5008
NAGG = 5120
NROWS = 2 * HALF
DUMP = NENT


def _mul_rows(acc_ref, b_ref, n=CH):
    def row(r, _):
        for c in range(D // L):
            sl = pl.ds(c * L, L)
            acc_ref[r, sl] = acc_ref[r, sl] * b_ref[r, sl]
        return 0
    lax.fori_loop(0, n, row, 0)


def _mul_rows_cvec(acc_ref, b_ref, cvec_ref):
    def row(r, _):
        for c in range(D // L):
            sl = pl.ds(c * L, L)
            acc_ref[r, sl] = acc_ref[r, sl] * b_ref[r, sl] * cvec_ref[sl]
        return 0
    lax.fori_loop(0, CH, row, 0)


def _mb_write(mb_out, mbw_v, sid, cid, val):
    mbw_v[pl.ds(0, L)] = jnp.zeros((L,), jnp.int32) + val
    pltpu.sync_copy(mbw_v, mb_out.at[sid, cid])


def _mb_read(mb_out, mbr_v, sid, cid):
    pltpu.sync_copy(mb_out.at[sid, 1 - cid], mbr_v)


def _hyper_body(R_h, E_h, aidx_h, cvec_h, h_out, mb_out,
                aidx_v, acc_v, buf0_v, buf1_v, cvec_v, mbw_v, mbr_v,
                semr, semb0, semb1, semst):
    cid = lax.axis_index("c")
    sid = lax.axis_index("s")
    _mb_write(mb_out, mbw_v, sid, cid, 0)
    pltpu.sync_copy(cvec_h, cvec_v)
    bufs = (buf0_v, buf1_v)
    semb = (semb0, semb1)

    def body(k, g):
        _mb_read(mb_out, mbr_v, sid, cid)
        oth = jnp.where(k == 0, 0, mbr_v[pl.ds(0, L)][0])
        active = g + oth < PCHA

        @pl.when(active)
        def _():
            _mb_write(mb_out, mbw_v, sid, cid, g + 1)
            j = jnp.where(cid == 0, g, PCHA - 1 - g)
            flat = sid * PCHA + j
            pltpu.sync_copy(aidx_h.at[flat], aidx_v)

            @pl.when(g > 0)
            def _():
                pltpu.make_async_copy(
                    acc_v, h_out.at[pl.ds(0, CH)], semst).wait()
            cpr = pltpu.async_copy(R_h.at[aidx_v.at[0]], acc_v, semr)
            cps = [None, None]
            cps[0] = pltpu.async_copy(E_h.at[aidx_v.at[1]], bufs[0], semb[0])
            cpr.wait()
            for i in range(1, 7):
                cur = (i - 1) % 2
                if i < 6:
                    cps[i % 2] = pltpu.async_copy(
                        E_h.at[aidx_v.at[i + 1]], bufs[i % 2], semb[i % 2])
                cps[cur].wait()
                if i < 6:
                    _mul_rows(acc_v, bufs[cur])
                else:
                    _mul_rows_cvec(acc_v, bufs[cur], cvec_v)
            pltpu.async_copy(acc_v, h_out.at[pl.ds(flat * CH, CH)], semst)
        return jnp.where(active, g + 1, g)

    lax.fori_loop(0, PCHA, body, jnp.int32(0))
    pltpu.make_async_copy(acc_v, h_out.at[pl.ds(0, CH)], semst).wait()


def _edge_body(h_h, E_h, etsd_h, agg2_out, mb_out,
               ring_v, mbw_v, mbr_v, h0, h1, h2, h3, e0, e1, e2, e3, agg_s,
               sh0, sh1, sh2, sh3, se0, se1, se2, se3):
    cid = lax.axis_index("c")
    sid = lax.axis_index("s")
    hbuf = (h0, h1, h2, h3)
    ebuf = (e0, e1, e2, e3)
    semh = (sh0, sh1, sh2, sh3)
    seme = (se0, se1, se2, se3)
    _mb_write(mb_out, mbw_v, sid, cid, 0)

    def zrow(r, _):
        for c in range(D // L):
            h0[r, pl.ds(c * L, L)] = jnp.zeros((L,), jnp.float32)
        return 0
    lax.fori_loop(0, CHB, zrow, 0)

    nz = NAGG // (NS * CHB)
    def zchunk(k, _):
        pltpu.sync_copy(h0, agg_s.at[pl.ds((sid * nz + k) * CHB, CHB)])
        return 0
    lax.fori_loop(0, nz, zchunk, 0)
    plsc.subcore_barrier()

    def localize(slot):
        for c4 in range(CHB // L):
            sl = pl.ds(c4 * L, L)
            d = ring_v[slot, 2, sl]
            loc = d - cid * HALF
            own = (loc >= 0) & (loc < HALF)
            ring_v[slot, 2, sl] = jnp.where(own, loc, HALF)

    base = sid * PCH + GRP

    def group_start(g):
        return jnp.where(cid == 0, base + GRP * g,
                         base + PCH - GRP * (g + 1))

    s0 = group_start(0)
    for k in range(DEPTH):
        pltpu.sync_copy(etsd_h.at[s0 + k], ring_v.at[k])
        localize(k)
        pltpu.async_copy(h_h.at[ring_v.at[k, 0]], hbuf[k], semh[k])
        pltpu.async_copy(E_h.at[ring_v.at[k, 1]], ebuf[k], seme[k])

    def body(q, g):
        _mb_read(mb_out, mbr_v, sid, cid)
        oth = jnp.where(q == 0, 0, mbr_v[pl.ds(0, L)][0])
        active = g + oth < NGRP

        @pl.when(active)
        def _():
            _mb_write(mb_out, mbw_v, sid, cid, g + 1)
            start = group_start(g)
            nstart = group_start(g + 1)
            for k in range(GRP):
                s = k % DEPTH
                pslot = (k + DEPTH) % GRP
                pltpu.make_async_copy(
                    h_h.at[ring_v.at[k, 0]], hbuf[s], semh[s]).wait()
                pltpu.make_async_copy(
                    E_h.at[ring_v.at[k, 1]], ebuf[s], seme[s]).wait()
                _mul_rows(hbuf[s], ebuf[s], CHB)
                pltpu.sync_copy(hbuf[s], agg_s.at[ring_v.at[k, 2]], add=True)
                pidx = (start + k + DEPTH) if k < GRP - DEPTH else (
                    nstart + k - (GRP - DEPTH))
                pltpu.sync_copy(etsd_h.at[pidx], ring_v.at[pslot])
                localize(pslot)
                pltpu.async_copy(h_h.at[ring_v.at[pslot, 0]], hbuf[s], semh[s])
                pltpu.async_copy(E_h.at[ring_v.at[pslot, 1]], ebuf[s], seme[s])
        return jnp.where(active, g + 1, g)

    lax.fori_loop(0, NGRP, body, jnp.int32(0))
    for k in range(DEPTH):
        pltpu.make_async_copy(h_h.at[ring_v.at[k, 0]], hbuf[k], semh[k]).wait()
        pltpu.make_async_copy(E_h.at[ring_v.at[k, 1]], ebuf[k], seme[k]).wait()
    plsc.subcore_barrier()

    no = NAGG // (NS * CHB)
    def ochunk(k, _):
        off = (sid * no + k) * CHB
        pltpu.sync_copy(agg_s.at[pl.ds(off, CHB)],
                        agg2_out.at[cid, pl.ds(off, CHB)])
        return 0
    lax.fori_loop(0, no, ochunk, 0)


def _post_body(agg2_ref, E_ref, R_ref, w_ref, g_ref, b_ref,
               out_ref, rout_ref, qc_ref):
    agg = jnp.concatenate([agg2_ref[0, :HALF], agg2_ref[1, :HALF]], axis=0)
    pre = agg * 0.5 + E_ref[...] * 0.5
    rows = lax.broadcasted_iota(jnp.int32, (NROWS, 1), 0)
    mask = rows < NENT
    xm = jnp.where(mask, pre, 0.0)
    s1 = jnp.sum(xm, axis=0, keepdims=True)
    s2 = jnp.sum(xm * xm, axis=0, keepdims=True)
    mean = s1 / NENT
    var = s2 / NENT - mean * mean
    inv = lax.rsqrt(var + 1e-5)
    y = jnp.tanh((pre - mean) * inv * g_ref[...] + b_ref[...])
    out_ref[...] = y
    rout_ref[...] = jnp.dot(R_ref[...], w_ref[...],
                            preferred_element_type=jnp.float32)
    qc = (y[ENT:ENT + 1] * y[ENT + 1:ENT + 2] * y[ENT + 2:ENT + 3]
          * y[ENT + 3:ENT + 4] * y[ENT + 4:ENT + 5] * y[ENT + 5:ENT + 6])
    qc_ref[...] = jnp.broadcast_to(qc, (8, D))


def _score_body(out_h, rout_h, idx_h, prod_out,
                idx_v, acc_v, buf0_v, buf1_v, semr, semb0, semb1):
    wid = lax.axis_index("c") * NS + lax.axis_index("s")
    base0 = wid * CH
    pltpu.sync_copy(idx_h.at[wid], idx_v)
    bufs = (buf0_v, buf1_v)
    semb = (semb0, semb1)
    cpr = pltpu.async_copy(rout_h.at[idx_v.at[0]], acc_v, semr)
    cps = [None, None]
    cps[0] = pltpu.async_copy(out_h.at[idx_v.at[1]], bufs[0], semb[0])
    cpr.wait()
    for i in range(1, 7):
        cur = (i - 1) % 2
        if i < 6:
            cps[i % 2] = pltpu.async_copy(
                out_h.at[idx_v.at[i + 1]], bufs[i % 2], semb[i % 2])
        cps[cur].wait()
        _mul_rows(acc_v, bufs[cur])
    pltpu.sync_copy(acc_v, prod_out.at[pl.ds(base0, CH)])


def _final_body(prod_ref, qc_ref, score_ref):
    score_ref[...] = jnp.sum(prod_ref[...] * qc_ref[0:1, :], axis=1)


@functools.cache
def _build_calls():
    mesh = plsc.VectorSubcoreMesh(core_axis_name="c", subcore_axis_name="s",
                                  num_cores=NC, num_subcores=NS)
    hyper_call = functools.partial(
        pl.kernel,
        out_type=[jax.ShapeDtypeStruct((NH_PAD, D), jnp.float32),
                  jax.ShapeDtypeStruct((NS, NC, L), jnp.int32)],
        mesh=mesh,
        scratch_types=[
            pltpu.VMEM((7, CH), jnp.int32),
            pltpu.VMEM((CH, D), jnp.float32),
            pltpu.VMEM((CH, D), jnp.float32),
            pltpu.VMEM((CH, D), jnp.float32),
            pltpu.VMEM((D,), jnp.float32),
            pltpu.VMEM((L,), jnp.int32),
            pltpu.VMEM((L,), jnp.int32),
            pltpu.SemaphoreType.DMA,
            pltpu.SemaphoreType.DMA,
            pltpu.SemaphoreType.DMA,
            pltpu.SemaphoreType.DMA,
        ],
    )(_hyper_body)

    edge_call = functools.partial(
        pl.kernel,
        out_type=[jax.ShapeDtypeStruct((NC, NAGG, D), jnp.float32),
                  jax.ShapeDtypeStruct((NS, NC, L), jnp.int32)],
        mesh=mesh,
        scratch_types=(
            [pltpu.VMEM((GRP, 3, CHB), jnp.int32),
             pltpu.VMEM((L,), jnp.int32),
             pltpu.VMEM((L,), jnp.int32)]
            + [pltpu.VMEM((CHB, D), jnp.float32) for _ in range(2 * DEPTH)]
            + [pltpu.VMEM_SHARED((NAGG, D), jnp.float32)]
            + [pltpu.SemaphoreType.DMA for _ in range(2 * DEPTH)]
        ),
    )(_edge_body)

    post_call = pl.pallas_call(
        _post_body,
        out_shape=[
            jax.ShapeDtypeStruct((NROWS, D), jnp.float32),
            jax.ShapeDtypeStruct((512, D), jnp.float32),
            jax.ShapeDtypeStruct((8, D), jnp.float32),
        ],
    )

    score_call = functools.partial(
        pl.kernel,
        out_type=jax.ShapeDtypeStruct((B, D), jnp.float32),
        mesh=mesh,
        scratch_types=[
            pltpu.VMEM((7, CH), jnp.int32),
            pltpu.VMEM((CH, D), jnp.float32),
            pltpu.VMEM((CH, D), jnp.float32),
            pltpu.VMEM((CH, D), jnp.float32),
            pltpu.SemaphoreType.DMA,
            pltpu.SemaphoreType.DMA,
            pltpu.SemaphoreType.DMA,
        ],
    )(_score_body)

    final_call = pl.pallas_call(
        _final_body,
        out_shape=jax.ShapeDtypeStruct((B,), jnp.float32),
    )
    return hyper_call, edge_call, post_call, score_call, final_call


def kernel(E, R, w_rel, bn_gamma, bn_beta, hyperedge, edge_index, edge_type,
           r_idx, e1_idx, e2_idx, e3_idx, e4_idx, e5_idx, e6_idx):
    f32 = jnp.float32
    i32 = jnp.int32
    E_pad = jnp.zeros((NROWS, D), f32).at[:NENT].set(E)
    R_pad = jnp.zeros((512, D), f32).at[:NREL].set(R)
    cvec = (E[ENT] * E[ENT + 1] * E[ENT + 2]
            * E[ENT + 3] * E[ENT + 4] * E[ENT + 5])

    nch_a = NW * HCH
    relidx = (jnp.zeros((NH_PAD,), i32).at[:NH].set(hyperedge[:, 0].astype(i32))
              .reshape(nch_a, 1, CH))
    entidx = (jnp.zeros((6, NH_PAD), i32)
              .at[:, :NH].set(hyperedge[:, 1:7].T.astype(i32))
              .reshape(6, nch_a, CH).transpose(1, 0, 2))
    aidx = jnp.concatenate([relidx, entidx], axis=1)

    et = jnp.zeros((NE_PAD,), i32).at[:NE].set(edge_type.astype(i32))
    src = jnp.zeros((NE_PAD,), i32).at[:NE].set(edge_index[1].astype(i32))
    dst = (jnp.full((NE_PAD,), DUMP, i32)
           .at[:NE].set(edge_index[0].astype(i32)))
    etsd = jnp.stack([et.reshape(NCHK, CHB), src.reshape(NCHK, CHB),
                      dst.reshape(NCHK, CHB)], axis=1)
    pad3 = jnp.zeros((GRP, 3, CHB), i32)
    etsd = jnp.concatenate([pad3, etsd, pad3], axis=0)

    idxpack = (jnp.stack([
        r_idx.astype(i32), e1_idx.astype(i32), e2_idx.astype(i32),
        e3_idx.astype(i32), e4_idx.astype(i32), e5_idx.astype(i32),
        e6_idx.astype(i32)]).reshape(7, NW, CH).transpose(1, 0, 2))

    hyper_call, edge_call, post_call, score_call, final_call = _build_calls()
    h, _mba = hyper_call(R, E_pad, aidx, cvec)
    agg2, _mbb = edge_call(h, E_pad, etsd)
    out, rout, qc = post_call(agg2, E_pad, R_pad, w_rel,
                              bn_gamma.reshape(1, D), bn_beta.reshape(1, D))
    prod = score_call(out, rout, idxpack)
    score = final_call(prod, qc)
    return score


# speedup vs baseline: 1.0126x; 1.0126x over previous
import functools

import jax
import jax.numpy as jnp
from jax import lax
from jax.experimental import pallas as pl
from jax.experimental.pallas import tpu as pltpu
from jax.experimental.pallas import tpu_sc as plsc

ENT = 10000
NENT = 10006
NREL = 500
NH = 20000
NE = 320000
D = 128
B = 4096
L = 16
NC, NS = 2, 16
NW = NC * NS
CH = 128
HCH = 5
NH_PAD = NW * HCH * CH
CHB = 64
PCH = 320
DEPTH = 3
GRP = 8
NGRP = PCH // GRP
NCHK = (NW // 2) * PCH
NE_PAD = NCHK * CHB
PCHA = 2 * HCH
HALF = 5008
NAGG = 5120
NROWS = 2 * HALF
DUMP = NENT


def _mul_rows(acc_ref, b_ref, n=CH):
    def row(r, _):
        for c in range(D // L):
            sl = pl.ds(c * L, L)
            acc_ref[r, sl] = acc_ref[r, sl] * b_ref[r, sl]
        return 0
    lax.fori_loop(0, n, row, 0)


def _mul_rows_cvec(acc_ref, b_ref, cvec_ref):
    def row(r, _):
        for c in range(D // L):
            sl = pl.ds(c * L, L)
            acc_ref[r, sl] = acc_ref[r, sl] * b_ref[r, sl] * cvec_ref[sl]
        return 0
    lax.fori_loop(0, CH, row, 0)


def _mb_write(mb_out, mbw_v, sid, cid, val):
    mbw_v[pl.ds(0, L)] = jnp.zeros((L,), jnp.int32) + val
    pltpu.sync_copy(mbw_v, mb_out.at[sid, cid])


def _mb_read(mb_out, mbr_v, sid, cid):
    pltpu.sync_copy(mb_out.at[sid, 1 - cid], mbr_v)


def _hyper_body(R_h, E_h, aidx_h, cvec_h, h_out, mb_out,
                aidx_v, acc_v, buf0_v, buf1_v, cvec_v, mbw_v, mbr_v,
                semr, semb0, semb1, semst):
    cid = lax.axis_index("c")
    sid = lax.axis_index("s")
    _mb_write(mb_out, mbw_v, sid, cid, 0)
    pltpu.sync_copy(cvec_h, cvec_v)
    bufs = (buf0_v, buf1_v)
    semb = (semb0, semb1)

    def body(k, g):
        _mb_read(mb_out, mbr_v, sid, cid)
        oth = jnp.where(k == 0, 0, mbr_v[pl.ds(0, L)][0])
        active = g + oth < PCHA

        @pl.when(active)
        def _():
            _mb_write(mb_out, mbw_v, sid, cid, g + 1)
            j = jnp.where(cid == 0, g, PCHA - 1 - g)
            flat = sid * PCHA + j
            pltpu.sync_copy(aidx_h.at[flat], aidx_v)

            @pl.when(g > 0)
            def _():
                pltpu.make_async_copy(
                    acc_v, h_out.at[pl.ds(0, CH)], semst).wait()
            cpr = pltpu.async_copy(R_h.at[aidx_v.at[0]], acc_v, semr)
            cps = [None, None]
            cps[0] = pltpu.async_copy(E_h.at[aidx_v.at[1]], bufs[0], semb[0])
            cpr.wait()
            for i in range(1, 7):
                cur = (i - 1) % 2
                if i < 6:
                    cps[i % 2] = pltpu.async_copy(
                        E_h.at[aidx_v.at[i + 1]], bufs[i % 2], semb[i % 2])
                cps[cur].wait()
                if i < 6:
                    _mul_rows(acc_v, bufs[cur])
                else:
                    _mul_rows_cvec(acc_v, bufs[cur], cvec_v)
            pltpu.async_copy(acc_v, h_out.at[pl.ds(flat * CH, CH)], semst)
        return jnp.where(active, g + 1, g)

    lax.fori_loop(0, PCHA, body, jnp.int32(0))
    pltpu.make_async_copy(acc_v, h_out.at[pl.ds(0, CH)], semst).wait()


def _mul_into(dst_ref, a_ref, b_ref, n):
    def row(r, _):
        for c in range(D // L):
            sl = pl.ds(c * L, L)
            dst_ref[r, sl] = a_ref[r, sl] * b_ref[r, sl]
        return 0
    lax.fori_loop(0, n, row, 0)


def _edge_body(h_h, E_h, etsd_h, agg2_out, mb_out,
               ring_v, mbw_v, mbr_v, h0, h1, h2, s0buf, e0, e1, e2, s1buf,
               agg_s, sh0, sh1, sh2, ssc0, se0, se1, se2, ssc1):
    cid = lax.axis_index("c")
    sid = lax.axis_index("s")
    hbuf = (h0, h1, h2)
    ebuf = (e0, e1, e2)
    sbuf = (s0buf, s1buf)
    semh = (sh0, sh1, sh2)
    seme = (se0, se1, se2)
    semsc = (ssc0, ssc1)
    _mb_write(mb_out, mbw_v, sid, cid, 0)

    def zrow(r, _):
        for c in range(D // L):
            h0[r, pl.ds(c * L, L)] = jnp.zeros((L,), jnp.float32)
        return 0
    lax.fori_loop(0, CHB, zrow, 0)

    nz = NAGG // (NS * CHB)
    def zchunk(k, _):
        pltpu.sync_copy(h0, agg_s.at[pl.ds((sid * nz + k) * CHB, CHB)])
        return 0
    lax.fori_loop(0, nz, zchunk, 0)
    plsc.subcore_barrier()

    def localize(slot):
        for c4 in range(CHB // L):
            sl = pl.ds(c4 * L, L)
            d = ring_v[slot, 2, sl]
            loc = d - cid * HALF
            own = (loc >= 0) & (loc < HALF)
            ring_v[slot, 2, sl] = jnp.where(own, loc, HALF)

    base = sid * PCH + GRP

    def group_start(g):
        return jnp.where(cid == 0, base + GRP * g,
                         base + PCH - GRP * (g + 1))

    s0 = group_start(0)
    for k in range(DEPTH):
        pltpu.sync_copy(etsd_h.at[s0 + k], ring_v.at[k])
        localize(k)
        pltpu.async_copy(h_h.at[ring_v.at[k, 0]], hbuf[k], semh[k])
        pltpu.async_copy(E_h.at[ring_v.at[k, 1]], ebuf[k], seme[k])

    def body(q, g):
        _mb_read(mb_out, mbr_v, sid, cid)
        oth = jnp.where(q == 0, 0, mbr_v[pl.ds(0, L)][0])
        active = g + oth < NGRP

        @pl.when(active)
        def _():
            _mb_write(mb_out, mbw_v, sid, cid, g + 1)
            start = group_start(g)
            nstart = group_start(g + 1)
            for k in range(GRP):
                s = k % DEPTH
                t = k % 2
                pslot = (k + DEPTH) % GRP

                def drain_sc(kk=k, tt=t):
                    pltpu.make_async_copy(
                        sbuf[tt], agg_s.at[ring_v.at[(kk - 2) % GRP, 2]],
                        semsc[tt]).wait()
                if k >= 2:
                    drain_sc()
                else:
                    pl.when(g > 0)(drain_sc)
                pltpu.make_async_copy(
                    h_h.at[ring_v.at[k, 0]], hbuf[s], semh[s]).wait()
                pltpu.make_async_copy(
                    E_h.at[ring_v.at[k, 1]], ebuf[s], seme[s]).wait()
                _mul_into(sbuf[t], hbuf[s], ebuf[s], CHB)
                pltpu.async_copy(sbuf[t], agg_s.at[ring_v.at[k, 2]],
                                 semsc[t], add=True)
                pidx = (start + k + DEPTH) if k < GRP - DEPTH else (
                    nstart + k - (GRP - DEPTH))
                pltpu.sync_copy(etsd_h.at[pidx], ring_v.at[pslot])
                localize(pslot)
                pltpu.async_copy(h_h.at[ring_v.at[pslot, 0]], hbuf[s], semh[s])
                pltpu.async_copy(E_h.at[ring_v.at[pslot, 1]], ebuf[s], seme[s])
        return jnp.where(active, g + 1, g)

    lax.fori_loop(0, NGRP, body, jnp.int32(0))
    for k in range(DEPTH):
        pltpu.make_async_copy(h_h.at[ring_v.at[k, 0]], hbuf[k], semh[k]).wait()
        pltpu.make_async_copy(E_h.at[ring_v.at[k, 1]], ebuf[k], seme[k]).wait()
    for t in range(2):
        pltpu.make_async_copy(
            sbuf[t], agg_s.at[ring_v.at[6 + t, 2]], semsc[t]).wait()
    plsc.subcore_barrier()

    no = NAGG // (NS * CHB)
    def ochunk(k, _):
        off = (sid * no + k) * CHB
        pltpu.sync_copy(agg_s.at[pl.ds(off, CHB)],
                        agg2_out.at[cid, pl.ds(off, CHB)])
        return 0
    lax.fori_loop(0, no, ochunk, 0)


def _post_body(agg2_ref, E_ref, R_ref, w_ref, g_ref, b_ref,
               out_ref, rout_ref, qc_ref):
    agg = jnp.concatenate([agg2_ref[0, :HALF], agg2_ref[1, :HALF]], axis=0)
    pre = agg * 0.5 + E_ref[...] * 0.5
    rows = lax.broadcasted_iota(jnp.int32, (NROWS, 1), 0)
    mask = rows < NENT
    xm = jnp.where(mask, pre, 0.0)
    s1 = jnp.sum(xm, axis=0, keepdims=True)
    s2 = jnp.sum(xm * xm, axis=0, keepdims=True)
    mean = s1 / NENT
    var = s2 / NENT - mean * mean
    inv = lax.rsqrt(var + 1e-5)
    y = jnp.tanh((pre - mean) * inv * g_ref[...] + b_ref[...])
    out_ref[...] = y
    rout_ref[...] = jnp.dot(R_ref[...], w_ref[...],
                            preferred_element_type=jnp.float32)
    qc = (y[ENT:ENT + 1] * y[ENT + 1:ENT + 2] * y[ENT + 2:ENT + 3]
          * y[ENT + 3:ENT + 4] * y[ENT + 4:ENT + 5] * y[ENT + 5:ENT + 6])
    qc_ref[...] = jnp.broadcast_to(qc, (8, D))


def _score_body(out_h, rout_h, idx_h, prod_out,
                idx_v, acc_v, buf0_v, buf1_v, semr, semb0, semb1):
    wid = lax.axis_index("c") * NS + lax.axis_index("s")
    base0 = wid * CH
    pltpu.sync_copy(idx_h.at[wid], idx_v)
    bufs = (buf0_v, buf1_v)
    semb = (semb0, semb1)
    cpr = pltpu.async_copy(rout_h.at[idx_v.at[0]], acc_v, semr)
    cps = [None, None]
    cps[0] = pltpu.async_copy(out_h.at[idx_v.at[1]], bufs[0], semb[0])
    cpr.wait()
    for i in range(1, 7):
        cur = (i - 1) % 2
        if i < 6:
            cps[i % 2] = pltpu.async_copy(
                out_h.at[idx_v.at[i + 1]], bufs[i % 2], semb[i % 2])
        cps[cur].wait()
        _mul_rows(acc_v, bufs[cur])
    pltpu.sync_copy(acc_v, prod_out.at[pl.ds(base0, CH)])


def _final_body(prod_ref, qc_ref, score_ref):
    score_ref[...] = jnp.sum(prod_ref[...] * qc_ref[0:1, :], axis=1)


@functools.cache
def _build_calls():
    mesh = plsc.VectorSubcoreMesh(core_axis_name="c", subcore_axis_name="s",
                                  num_cores=NC, num_subcores=NS)
    hyper_call = functools.partial(
        pl.kernel,
        out_type=[jax.ShapeDtypeStruct((NH_PAD, D), jnp.float32),
                  jax.ShapeDtypeStruct((NS, NC, L), jnp.int32)],
        mesh=mesh,
        scratch_types=[
            pltpu.VMEM((7, CH), jnp.int32),
            pltpu.VMEM((CH, D), jnp.float32),
            pltpu.VMEM((CH, D), jnp.float32),
            pltpu.VMEM((CH, D), jnp.float32),
            pltpu.VMEM((D,), jnp.float32),
            pltpu.VMEM((L,), jnp.int32),
            pltpu.VMEM((L,), jnp.int32),
            pltpu.SemaphoreType.DMA,
            pltpu.SemaphoreType.DMA,
            pltpu.SemaphoreType.DMA,
            pltpu.SemaphoreType.DMA,
        ],
    )(_hyper_body)

    edge_call = functools.partial(
        pl.kernel,
        out_type=[jax.ShapeDtypeStruct((NC, NAGG, D), jnp.float32),
                  jax.ShapeDtypeStruct((NS, NC, L), jnp.int32)],
        mesh=mesh,
        scratch_types=(
            [pltpu.VMEM((GRP, 3, CHB), jnp.int32),
             pltpu.VMEM((L,), jnp.int32),
             pltpu.VMEM((L,), jnp.int32)]
            + [pltpu.VMEM((CHB, D), jnp.float32) for _ in range(2 * DEPTH + 2)]
            + [pltpu.VMEM_SHARED((NAGG, D), jnp.float32)]
            + [pltpu.SemaphoreType.DMA for _ in range(2 * DEPTH + 2)]
        ),
    )(_edge_body)

    post_call = pl.pallas_call(
        _post_body,
        out_shape=[
            jax.ShapeDtypeStruct((NROWS, D), jnp.float32),
            jax.ShapeDtypeStruct((512, D), jnp.float32),
            jax.ShapeDtypeStruct((8, D), jnp.float32),
        ],
    )

    score_call = functools.partial(
        pl.kernel,
        out_type=jax.ShapeDtypeStruct((B, D), jnp.float32),
        mesh=mesh,
        scratch_types=[
            pltpu.VMEM((7, CH), jnp.int32),
            pltpu.VMEM((CH, D), jnp.float32),
            pltpu.VMEM((CH, D), jnp.float32),
            pltpu.VMEM((CH, D), jnp.float32),
            pltpu.SemaphoreType.DMA,
            pltpu.SemaphoreType.DMA,
            pltpu.SemaphoreType.DMA,
        ],
    )(_score_body)

    final_call = pl.pallas_call(
        _final_body,
        out_shape=jax.ShapeDtypeStruct((B,), jnp.float32),
    )
    return hyper_call, edge_call, post_call, score_call, final_call


def kernel(E, R, w_rel, bn_gamma, bn_beta, hyperedge, edge_index, edge_type,
           r_idx, e1_idx, e2_idx, e3_idx, e4_idx, e5_idx, e6_idx):
    f32 = jnp.float32
    i32 = jnp.int32
    E_pad = jnp.zeros((NROWS, D), f32).at[:NENT].set(E)
    R_pad = jnp.zeros((512, D), f32).at[:NREL].set(R)
    cvec = (E[ENT] * E[ENT + 1] * E[ENT + 2]
            * E[ENT + 3] * E[ENT + 4] * E[ENT + 5])

    nch_a = NW * HCH
    relidx = (jnp.zeros((NH_PAD,), i32).at[:NH].set(hyperedge[:, 0].astype(i32))
              .reshape(nch_a, 1, CH))
    entidx = (jnp.zeros((6, NH_PAD), i32)
              .at[:, :NH].set(hyperedge[:, 1:7].T.astype(i32))
              .reshape(6, nch_a, CH).transpose(1, 0, 2))
    aidx = jnp.concatenate([relidx, entidx], axis=1)

    et = jnp.zeros((NE_PAD,), i32).at[:NE].set(edge_type.astype(i32))
    src = jnp.zeros((NE_PAD,), i32).at[:NE].set(edge_index[1].astype(i32))
    dst = (jnp.full((NE_PAD,), DUMP, i32)
           .at[:NE].set(edge_index[0].astype(i32)))
    etsd = jnp.stack([et.reshape(NCHK, CHB), src.reshape(NCHK, CHB),
                      dst.reshape(NCHK, CHB)], axis=1)
    pad3 = jnp.zeros((GRP, 3, CHB), i32)
    etsd = jnp.concatenate([pad3, etsd, pad3], axis=0)

    idxpack = (jnp.stack([
        r_idx.astype(i32), e1_idx.astype(i32), e2_idx.astype(i32),
        e3_idx.astype(i32), e4_idx.astype(i32), e5_idx.astype(i32),
        e6_idx.astype(i32)]).reshape(7, NW, CH).transpose(1, 0, 2))

    hyper_call, edge_call, post_call, score_call, final_call = _build_calls()
    h, _mba = hyper_call(R, E_pad, aidx, cvec)
    agg2, _mbb = edge_call(h, E_pad, etsd)
    out, rout, qc = post_call(agg2, E_pad, R_pad, w_rel,
                              bn_gamma.reshape(1, D), bn_beta.reshape(1, D))
    prod = score_call(out, rout, idxpack)
    score = final_call(prod, qc)
    return score
